# baseline (speedup 1.0000x reference)
import jax
import jax.numpy as jnp
from jax import lax
from jax.experimental import pallas as pl
from jax.experimental.pallas import tpu as pltpu
from jax.experimental.pallas import tpu_sc as plsc

N = 10000
E = 320000
NPAD = 10240
TRASH = N
ROWS_PER_TILE = NPAD // 16
CHUNK = 128
SUB = 2
BATCH = CHUNK * SUB
HR = NPAD // 128


def _make_agg(nb, sub, chunk):
    mesh = plsc.VectorSubcoreMesh(core_axis_name="c", subcore_axis_name="s",
                                  num_cores=2, num_subcores=16)

    def body(table, edges, zeros, out, ed_v, rows_v, acc, sem_g, sem_s):
        c = lax.axis_index("c")
        s = lax.axis_index("s")
        r0 = s * ROWS_PER_TILE
        pltpu.sync_copy(zeros.at[pl.ds(r0, ROWS_PER_TILE)],
                        acc.at[pl.ds(r0, ROWS_PER_TILE)])
        plsc.subcore_barrier()

        def _drain_puts():
            for j in range(sub):
                pltpu.make_async_copy(table.at[pl.ds(0, chunk)],
                                      rows_v.at[j], sem_s).wait()

        def step(k, carry):
            b = s * nb + k
            par = lax.rem(k, 2)
            pltpu.sync_copy(edges.at[c, b], ed_v.at[par])

            @pl.when(k > 0)
            def _():
                _drain_puts()

            gets = [pltpu.async_copy(table.at[ed_v.at[par, 0, j]],
                                     rows_v.at[j], sem_g)
                    for j in range(sub)]
            for j in range(sub):
                gets[j].wait()
                pltpu.async_copy(rows_v.at[j], acc.at[ed_v.at[par, 1, j]],
                                 sem_s, add=True)
            return carry

        lax.fori_loop(0, nb, step, 0)
        _drain_puts()
        plsc.subcore_barrier()
        pltpu.sync_copy(acc.at[pl.ds(r0, ROWS_PER_TILE)],
                        out.at[c, pl.ds(r0, ROWS_PER_TILE)])

    return pl.kernel(
        body,
        out_type=jax.ShapeDtypeStruct((2, NPAD, 128), jnp.float32),
        mesh=mesh,
        scratch_types=[
            pltpu.VMEM((2, 2, sub, chunk), jnp.int32),
            pltpu.VMEM((sub, chunk, 128), jnp.float32),
            pltpu.VMEM_SHARED((NPAD, 128), jnp.float32),
            pltpu.SemaphoreType.DMA,
            pltpu.SemaphoreType.DMA,
        ],
        compiler_params=pltpu.CompilerParams(needs_layout_passes=False))


_DEG_ROWS = 80


def _make_deg():
    mesh = plsc.VectorSubcoreMesh(core_axis_name="c", subcore_axis_name="s",
                                  num_cores=2, num_subcores=16)
    nb = _DEG_ROWS // 8

    def body(dsth, zeros, deg_out, dv, hist, ones_v):
        c = lax.axis_index("c")
        s = lax.axis_index("s")
        pltpu.sync_copy(zeros.at[pl.ds(0, HR)], hist)

        def step(k, carry):
            pltpu.sync_copy(dsth.at[c, pl.ds(s * _DEG_ROWS + k * 8, 8)], dv)
            ones16 = jnp.ones((16,), jnp.float32)
            for j in range(8):
                for i in range(8):
                    v = dv[j, pl.ds(i * 16, 16)]
                    plsc.addupdate_scatter(
                        hist,
                        [lax.shift_right_logical(v, 7),
                         lax.bitwise_and(v, 127)],
                        ones16)
            return carry

        lax.fori_loop(0, nb, step, 0)
        pltpu.sync_copy(hist, deg_out.at[c, s])

    return pl.kernel(
        body,
        out_type=jax.ShapeDtypeStruct((2, 16, HR, 128), jnp.float32),
        mesh=mesh,
        scratch_types=[
            pltpu.VMEM((8, 128), jnp.int32),
            pltpu.VMEM((HR, 128), jnp.float32),
            pltpu.VMEM((16,), jnp.float32),
        ],
        compiler_params=pltpu.CompilerParams(needs_layout_passes=False))


R = 1024
GRID = NPAD // R


def _widen(r88):
    parts = [jnp.broadcast_to(r88[r:r + 1, :], (128, 128)).T
             for r in range(8)]
    return jnp.concatenate(parts, axis=0)


def _tc1_body(p_ref, degp_ref, z_ref, w1lt_ref, w1rt_ref, b1_ref,
              h1s_ref, rdeg_ref):
    deg = jnp.sum(degp_ref[...], axis=(0, 1))
    rdeg = 1.0 / jnp.maximum(deg, 1.0)
    agg = (p_ref[0] + p_ref[1]) * _widen(rdeg)
    h1 = jnp.dot(agg, w1lt_ref[...], preferred_element_type=jnp.float32)
    h1 += jnp.dot(z_ref[...], w1rt_ref[...], preferred_element_type=jnp.float32)
    h1 = jnp.maximum(h1 + b1_ref[...], 0.0)
    h1s_ref[0] = h1[:, :128]
    h1s_ref[1] = h1[:, 128:]
    rdeg_ref[...] = rdeg


def _tc2_body(p_ref, rdeg_ref, h1s_ref, w2lat_ref, w2lbt_ref, w2rat_ref,
              w2rbt_ref, b2_ref, w3lt_ref, h2_ref, y3_ref):
    rdeg = _widen(rdeg_ref[...])
    h2 = jnp.dot(p_ref[0] * rdeg, w2lat_ref[...], preferred_element_type=jnp.float32)
    h2 += jnp.dot(p_ref[1] * rdeg, w2lbt_ref[...], preferred_element_type=jnp.float32)
    h2 += jnp.dot(h1s_ref[0], w2rat_ref[...], preferred_element_type=jnp.float32)
    h2 += jnp.dot(h1s_ref[1], w2rbt_ref[...], preferred_element_type=jnp.float32)
    h2 = jnp.maximum(h2 + b2_ref[...], 0.0)
    h2_ref[...] = h2
    y3_ref[...] = jnp.dot(h2, w3lt_ref[...], preferred_element_type=jnp.float32)


def _tc3_body(p_ref, rdeg_ref, h2_ref, w3rt_ref, b3_ref, out_ref):
    agg = (p_ref[0] + p_ref[1]) * _widen(rdeg_ref[...])
    o = agg + jnp.dot(h2_ref[...], w3rt_ref[...], preferred_element_type=jnp.float32)
    out_ref[...] = jnp.maximum(o + b3_ref[...], 0.0)


def _full(shape):
    return pl.BlockSpec(shape, lambda i: (0,) * len(shape))


def _rows(shape):
    nd = len(shape)
    return pl.BlockSpec(shape, lambda i, _nd=nd: (0,) * (_nd - 2) + (i, 0))


_tc1 = pl.pallas_call(
    _tc1_body,
    grid=(GRID,),
    in_specs=[
        _rows((2, R, 128)),
        pl.BlockSpec((2, 16, R // 128, 128), lambda i: (0, 0, i, 0)),
        _rows((R, 128)),
        _full((128, 256)),
        _full((128, 256)),
        _full((1, 256)),
    ],
    out_specs=[
        _rows((2, R, 128)),
        pl.BlockSpec((R // 128, 128), lambda i: (i, 0)),
    ],
    out_shape=[
        jax.ShapeDtypeStruct((2, NPAD, 128), jnp.float32),
        jax.ShapeDtypeStruct((NPAD // 128, 128), jnp.float32),
    ],
)

_tc2 = pl.pallas_call(
    _tc2_body,
    grid=(GRID,),
    in_specs=[
        _rows((2, R, 128)),
        pl.BlockSpec((R // 128, 128), lambda i: (i, 0)),
        _rows((2, R, 128)),
        _full((128, 256)),
        _full((128, 256)),
        _full((128, 256)),
        _full((128, 256)),
        _full((1, 256)),
        _full((256, 128)),
    ],
    out_specs=[
        _rows((R, 256)),
        _rows((R, 128)),
    ],
    out_shape=[
        jax.ShapeDtypeStruct((NPAD, 256), jnp.float32),
        jax.ShapeDtypeStruct((NPAD, 128), jnp.float32),
    ],
)

_tc3 = pl.pallas_call(
    _tc3_body,
    grid=(GRID,),
    in_specs=[
        _rows((2, R, 128)),
        pl.BlockSpec((R // 128, 128), lambda i: (i, 0)),
        _rows((R, 256)),
        _full((256, 128)),
        _full((1, 128)),
    ],
    out_specs=_rows((R, 128)),
    out_shape=jax.ShapeDtypeStruct((NPAD, 128), jnp.float32),
)

_NB_SPLIT = 28
_NB_COL = 56
_agg_split = _make_agg(_NB_SPLIT, 3, 120)
_agg_col = _make_agg(_NB_COL, 3, 120)
_deg_kernel = _make_deg()


def _pad_edges(a, total, fill):
    return jnp.concatenate([a, jnp.full((total - a.shape[0],), fill, jnp.int32)])


def kernel(z, edge_index, W1l, W1r, b1, W2l, W2r, b2, W3l, W3r, b3):
    src = edge_index[0]
    dst = edge_index[1]

    e_half = E // 2

    def _blocks(s_arr, d_arr, nb, sub, chunk):
        tot = 16 * nb * sub * chunk
        s_p = _pad_edges(s_arr, tot, 0)
        d_p = _pad_edges(d_arr, tot, TRASH)
        return jnp.stack([s_p.reshape(-1, sub, chunk),
                          d_p.reshape(-1, sub, chunk)], axis=1)

    ed_split = jnp.stack([_blocks(src[:e_half], dst[:e_half], _NB_SPLIT, 3, 120),
                          _blocks(src[e_half:], dst[e_half:], _NB_SPLIT, 3, 120)])
    ed_col = jnp.stack([_blocks(src, dst, _NB_COL, 3, 120),
                        _blocks(src + NPAD, dst, _NB_COL, 3, 120)])
    dh_tot = 16 * _DEG_ROWS * 128
    dh = jnp.stack([_pad_edges(dst[:e_half], dh_tot, TRASH).reshape(-1, 128),
                    _pad_edges(dst[e_half:], dh_tot, TRASH).reshape(-1, 128)])

    z_pad = jnp.concatenate([z, jnp.zeros((NPAD - N, 128), jnp.float32)])
    zeros128 = jnp.zeros((NPAD, 128), jnp.float32)

    w1lt = W1l.T
    w1rt = W1r.T
    w2lat = W2l[:, :128].T
    w2lbt = W2l[:, 128:].T
    w2rat = W2r[:, :128].T
    w2rbt = W2r[:, 128:].T
    w3lt = W3l.T
    w3rt = W3r.T

    degp = _deg_kernel(dh, zeros128)
    p1 = _agg_split(z_pad, ed_split, zeros128)
    h1s, rdeg = _tc1(p1, degp, z_pad, w1lt, w1rt, b1.reshape(1, 256))

    table2 = h1s.reshape(2 * NPAD, 128)
    p2 = _agg_col(table2, ed_col, zeros128)
    h2, y3 = _tc2(p2, rdeg, h1s, w2lat, w2lbt, w2rat, w2rbt,
                  b2.reshape(1, 256), w3lt)

    p3 = _agg_split(y3, ed_split, zeros128)
    out = _tc3(p3, rdeg, h2, w3rt, b3.reshape(1, 128))
    return out[:N]


# speedup vs baseline: 1.0002x; 1.0002x over previous
import jax
import jax.numpy as jnp
from jax import lax
from jax.experimental import pallas as pl
from jax.experimental.pallas import tpu as pltpu
from jax.experimental.pallas import tpu_sc as plsc

N = 10000
E = 320000
NPAD = 10240
TRASH = N
ROWS_PER_TILE = NPAD // 16
HR = NPAD // 128


def _make_agg(nb, sub, chunk):
    mesh = plsc.VectorSubcoreMesh(core_axis_name="c", subcore_axis_name="s",
                                  num_cores=2, num_subcores=16)

    def body(table, edges, zeros, out, ed_v, rows_v, acc, sem_g, sem_s):
        c = lax.axis_index("c")
        s = lax.axis_index("s")
        r0 = s * ROWS_PER_TILE
        pltpu.sync_copy(zeros.at[pl.ds(r0, ROWS_PER_TILE)],
                        acc.at[pl.ds(r0, ROWS_PER_TILE)])
        plsc.subcore_barrier()

        def _drain_puts():
            for j in range(sub):
                pltpu.make_async_copy(table.at[pl.ds(0, chunk)],
                                      rows_v.at[j], sem_s).wait()

        def step(k, carry):
            b = s * nb + k
            par = lax.rem(k, 2)
            pltpu.sync_copy(edges.at[c, b], ed_v.at[par])

            @pl.when(k > 0)
            def _():
                _drain_puts()

            gets = [pltpu.async_copy(table.at[ed_v.at[par, 0, j]],
                                     rows_v.at[j], sem_g)
                    for j in range(sub)]
            for j in range(sub):
                gets[j].wait()
                pltpu.async_copy(rows_v.at[j], acc.at[ed_v.at[par, 1, j]],
                                 sem_s, add=True)
            return carry

        lax.fori_loop(0, nb, step, 0)
        _drain_puts()
        plsc.subcore_barrier()
        pltpu.sync_copy(acc.at[pl.ds(r0, ROWS_PER_TILE)],
                        out.at[c, pl.ds(r0, ROWS_PER_TILE)])

    return pl.kernel(
        body,
        out_type=jax.ShapeDtypeStruct((2, NPAD, 128), jnp.float32),
        mesh=mesh,
        scratch_types=[
            pltpu.VMEM((2, 2, sub, chunk), jnp.int32),
            pltpu.VMEM((sub, chunk, 128), jnp.float32),
            pltpu.VMEM_SHARED((NPAD, 128), jnp.float32),
            pltpu.SemaphoreType.DMA,
            pltpu.SemaphoreType.DMA,
        ],
        compiler_params=pltpu.CompilerParams(needs_layout_passes=False))


_DEG_ROWS = 80


def _make_deg():
    mesh = plsc.VectorSubcoreMesh(core_axis_name="c", subcore_axis_name="s",
                                  num_cores=2, num_subcores=16)
    nb = _DEG_ROWS // 8

    def body(dsth, zeros, deg_out, dv, hist):
        c = lax.axis_index("c")
        s = lax.axis_index("s")
        pltpu.sync_copy(zeros.at[pl.ds(0, HR)], hist)

        def step(k, carry):
            pltpu.sync_copy(dsth.at[c, pl.ds(s * _DEG_ROWS + k * 8, 8)], dv)
            ones16 = jnp.ones((16,), jnp.float32)
            for j in range(8):
                for i in range(8):
                    v = dv[j, pl.ds(i * 16, 16)]
                    plsc.addupdate_scatter(
                        hist,
                        [lax.shift_right_logical(v, 7),
                         lax.bitwise_and(v, 127)],
                        ones16)
            return carry

        lax.fori_loop(0, nb, step, 0)
        pltpu.sync_copy(hist, deg_out.at[c, s])

    return pl.kernel(
        body,
        out_type=jax.ShapeDtypeStruct((2, 16, HR, 128), jnp.float32),
        mesh=mesh,
        scratch_types=[
            pltpu.VMEM((8, 128), jnp.int32),
            pltpu.VMEM((HR, 128), jnp.float32),
        ],
        compiler_params=pltpu.CompilerParams(needs_layout_passes=False))


R = 1024
GRID = NPAD // R


def _widen(r88):
    parts = [jnp.broadcast_to(r88[r:r + 1, :], (128, 128)).T
             for r in range(8)]
    return jnp.concatenate(parts, axis=0)


def _tc1_body(p_ref, degp_ref, z_ref, w1lt_ref, w1rt_ref, b1_ref,
              h1s_ref, rdeg_ref):
    deg = jnp.sum(degp_ref[...], axis=(0, 1))
    rdeg = 1.0 / jnp.maximum(deg, 1.0)
    agg = (p_ref[0] + p_ref[1]) * _widen(rdeg)
    h1 = jnp.dot(agg, w1lt_ref[...], preferred_element_type=jnp.float32)
    h1 += jnp.dot(z_ref[...], w1rt_ref[...], preferred_element_type=jnp.float32)
    h1 = jnp.maximum(h1 + b1_ref[...], 0.0)
    h1s_ref[0] = h1[:, :128]
    h1s_ref[1] = h1[:, 128:]
    rdeg_ref[...] = rdeg


def _tc2_body(p_ref, rdeg_ref, h1s_ref, w2lat_ref, w2lbt_ref, w2rat_ref,
              w2rbt_ref, b2_ref, w3lt_ref, h2_ref, y3_ref):
    rdeg = _widen(rdeg_ref[...])
    h2 = jnp.dot(p_ref[0] * rdeg, w2lat_ref[...], preferred_element_type=jnp.float32)
    h2 += jnp.dot(p_ref[1] * rdeg, w2lbt_ref[...], preferred_element_type=jnp.float32)
    h2 += jnp.dot(h1s_ref[0], w2rat_ref[...], preferred_element_type=jnp.float32)
    h2 += jnp.dot(h1s_ref[1], w2rbt_ref[...], preferred_element_type=jnp.float32)
    h2 = jnp.maximum(h2 + b2_ref[...], 0.0)
    h2_ref[...] = h2
    y3_ref[...] = jnp.dot(h2, w3lt_ref[...], preferred_element_type=jnp.float32)


def _tc3_body(p_ref, rdeg_ref, h2_ref, w3rt_ref, b3_ref, out_ref):
    agg = (p_ref[0] + p_ref[1]) * _widen(rdeg_ref[...])
    o = agg + jnp.dot(h2_ref[...], w3rt_ref[...], preferred_element_type=jnp.float32)
    out_ref[...] = jnp.maximum(o + b3_ref[...], 0.0)


def _full(shape):
    return pl.BlockSpec(shape, lambda i: (0,) * len(shape))


def _rows(shape):
    nd = len(shape)
    return pl.BlockSpec(shape, lambda i, _nd=nd: (0,) * (_nd - 2) + (i, 0))


_tc1 = pl.pallas_call(
    _tc1_body,
    grid=(GRID,),
    in_specs=[
        _rows((2, R, 128)),
        pl.BlockSpec((2, 16, R // 128, 128), lambda i: (0, 0, i, 0)),
        _rows((R, 128)),
        _full((128, 256)),
        _full((128, 256)),
        _full((1, 256)),
    ],
    out_specs=[
        _rows((2, R, 128)),
        pl.BlockSpec((R // 128, 128), lambda i: (i, 0)),
    ],
    out_shape=[
        jax.ShapeDtypeStruct((2, NPAD, 128), jnp.float32),
        jax.ShapeDtypeStruct((NPAD // 128, 128), jnp.float32),
    ],
)

_tc2 = pl.pallas_call(
    _tc2_body,
    grid=(GRID,),
    in_specs=[
        _rows((2, R, 128)),
        pl.BlockSpec((R // 128, 128), lambda i: (i, 0)),
        _rows((2, R, 128)),
        _full((128, 256)),
        _full((128, 256)),
        _full((128, 256)),
        _full((128, 256)),
        _full((1, 256)),
        _full((256, 128)),
    ],
    out_specs=[
        _rows((R, 256)),
        _rows((R, 128)),
    ],
    out_shape=[
        jax.ShapeDtypeStruct((NPAD, 256), jnp.float32),
        jax.ShapeDtypeStruct((NPAD, 128), jnp.float32),
    ],
)

_tc3 = pl.pallas_call(
    _tc3_body,
    grid=(GRID,),
    in_specs=[
        _rows((2, R, 128)),
        pl.BlockSpec((R // 128, 128), lambda i: (i, 0)),
        _rows((R, 256)),
        _full((256, 128)),
        _full((1, 128)),
    ],
    out_specs=_rows((R, 128)),
    out_shape=jax.ShapeDtypeStruct((NPAD, 128), jnp.float32),
)

_NB_SPLIT = 28
_NB_COL = 56
_agg_split = _make_agg(_NB_SPLIT, 3, 120)
_agg_col = _make_agg(_NB_COL, 3, 120)
_deg_kernel = _make_deg()


def _pad_edges(a, total, fill):
    return jnp.concatenate([a, jnp.full((total - a.shape[0],), fill, jnp.int32)])


def kernel(z, edge_index, W1l, W1r, b1, W2l, W2r, b2, W3l, W3r, b3):
    src = edge_index[0]
    dst = edge_index[1]

    e_half = E // 2

    def _blocks(s_arr, d_arr, nb, sub, chunk):
        tot = 16 * nb * sub * chunk
        s_p = _pad_edges(s_arr, tot, 0)
        d_p = _pad_edges(d_arr, tot, TRASH)
        return jnp.stack([s_p.reshape(-1, sub, chunk),
                          d_p.reshape(-1, sub, chunk)], axis=1)

    ed_split = jnp.stack([_blocks(src[:e_half], dst[:e_half], _NB_SPLIT, 3, 120),
                          _blocks(src[e_half:], dst[e_half:], _NB_SPLIT, 3, 120)])
    ed_col = jnp.stack([_blocks(src, dst, _NB_COL, 3, 120),
                        _blocks(src + NPAD, dst, _NB_COL, 3, 120)])
    dh_tot = 16 * _DEG_ROWS * 128
    dh = jnp.stack([_pad_edges(dst[:e_half], dh_tot, TRASH).reshape(-1, 128),
                    _pad_edges(dst[e_half:], dh_tot, TRASH).reshape(-1, 128)])

    z_pad = jnp.concatenate([z, jnp.zeros((NPAD - N, 128), jnp.float32)])
    zeros128 = jnp.zeros((NPAD, 128), jnp.float32)

    w1lt = W1l.T
    w1rt = W1r.T
    w2lat = W2l[:, :128].T
    w2lbt = W2l[:, 128:].T
    w2rat = W2r[:, :128].T
    w2rbt = W2r[:, 128:].T
    w3lt = W3l.T
    w3rt = W3r.T

    degp = _deg_kernel(dh, zeros128)
    p1 = _agg_split(z_pad, ed_split, zeros128)
    h1s, rdeg = _tc1(p1, degp, z_pad, w1lt, w1rt, b1.reshape(1, 256))

    table2 = h1s.reshape(2 * NPAD, 128)
    p2 = _agg_col(table2, ed_col, zeros128)
    h2, y3 = _tc2(p2, rdeg, h1s, w2lat, w2lbt, w2rat, w2rbt,
                  b2.reshape(1, 256), w3lt)

    p3 = _agg_split(y3, ed_split, zeros128)
    out = _tc3(p3, rdeg, h2, w3rt, b3.reshape(1, 128))
    return out[:N]
